# Initial kernel scaffold; baseline (speedup 1.0000x reference)
import jax
import jax.numpy as jnp
from jax.experimental import pallas as pl


def kernel(x, Wg, W1, W3, W2):
    raise NotImplementedError("write your pallas kernel here")


# speedup vs baseline: 1.3786x; 1.3786x over previous
import functools
import jax
import jax.numpy as jnp
from jax.experimental import pallas as pl
from jax.experimental.pallas import tpu as pltpu

E = 8
TOPK = 2
NEG_INF = -1e30


def _router_body(x_ref, wg_ref, logits_ref, coeff_ref):
    x = x_ref[...]
    wg = wg_ref[...]
    logits = jnp.dot(x, wg, preferred_element_type=jnp.float32)
    logits_ref[...] = logits
    L = logits.shape[0]
    iota = jax.lax.broadcasted_iota(jnp.int32, (L, E), 1)
    m1 = jnp.max(logits, axis=1, keepdims=True)
    a1 = jnp.min(jnp.where(logits == m1, iota, E), axis=1, keepdims=True)
    masked = jnp.where(iota == a1, NEG_INF, logits)
    m2 = jnp.max(masked, axis=1, keepdims=True)
    a2 = jnp.min(jnp.where(masked == m2, iota, E), axis=1, keepdims=True)
    w1 = 1.0 / (1.0 + jnp.exp(m2 - m1))
    w2 = 1.0 - w1
    coeff_ref[...] = jnp.where(iota == a1, w1, 0.0) + jnp.where(iota == a2, w2, 0.0)


def _expert_body(x_ref, w1_ref, w3_ref, w2_ref, coeff_ref, out_ref):
    e = pl.program_id(0)
    f = pl.program_id(1)
    x = x_ref[...]
    L = x.shape[0]
    iota = jax.lax.broadcasted_iota(jnp.int32, (L, E), 1)
    col = jnp.sum(jnp.where(iota == e, coeff_ref[...], 0.0), axis=1,
                  keepdims=True)

    @pl.when((e == 0) & (f == 0))
    def _():
        out_ref[...] = jnp.zeros_like(out_ref)

    g = jnp.dot(x, w1_ref[0], preferred_element_type=jnp.float32)
    u = jnp.dot(x, w3_ref[0], preferred_element_type=jnp.float32)
    h = (g * jax.nn.sigmoid(g)) * u
    out_ref[...] += jnp.dot(h, w2_ref[0], preferred_element_type=jnp.float32) * col


def kernel(x, Wg, W1, W3, W2):
    Bs, L, D = x.shape
    x_flat = x.reshape(L, D)
    F = W1.shape[2]

    logits, coeff = pl.pallas_call(
        _router_body,
        out_shape=(
            jax.ShapeDtypeStruct((L, E), jnp.float32),
            jax.ShapeDtypeStruct((L, E), jnp.float32),
        ),
    )(x_flat, Wg)

    FC = 512
    out = pl.pallas_call(
        _expert_body,
        grid=(E, F // FC),
        in_specs=[
            pl.BlockSpec((L, D), lambda e, f: (0, 0)),
            pl.BlockSpec((1, D, FC), lambda e, f: (e, 0, f)),
            pl.BlockSpec((1, D, FC), lambda e, f: (e, 0, f)),
            pl.BlockSpec((1, FC, D), lambda e, f: (e, f, 0)),
            pl.BlockSpec((L, E), lambda e, f: (0, 0)),
        ],
        out_specs=pl.BlockSpec((L, D), lambda e, f: (0, 0)),
        out_shape=jax.ShapeDtypeStruct((L, D), jnp.float32),
        compiler_params=pltpu.CompilerParams(
            dimension_semantics=("arbitrary", "arbitrary"),
        ),
    )(x_flat, W1, W3, W2, coeff)

    return out.reshape(Bs, L, D), logits
